# baseline (speedup 1.0000x reference)
import functools

import jax
import jax.numpy as jnp
from jax import lax
from jax.experimental import pallas as pl
from jax.experimental.pallas import tpu as pltpu
from jax.experimental.pallas import tpu_sc as plsc

N = 10000
D = 128
H = 128
E = 320000
G = 1024
EPS = 1e-5
BNS = (1.0 + EPS) ** -0.5

NC = 2
NS = 16
NW = NC * NS
CHUNK = 128
CPT = 80
NDUM = 6
CPT2 = CPT + 2
NP = CPT2 // 2
EPT = CPT * CHUNK
EPAD = NW * EPT
NROWS = 10112
ZPS = NROWS // NS
NR_DEG = 10240
ZPS_DEG = NR_DEG // NS
GPT = G // NW

RB = 2000
NBLK = N // RB

_sc_mesh = plsc.VectorSubcoreMesh(
    core_axis_name="c", subcore_axis_name="s", num_cores=NC, num_subcores=NS
)


@functools.partial(
    pl.kernel,
    out_type=jax.ShapeDtypeStruct((NC, NR_DEG), jnp.float32),
    mesh=_sc_mesh,
    scratch_types=[
        pltpu.VMEM((CPT + NDUM, CHUNK), jnp.int32),
        pltpu.VMEM((CHUNK,), jnp.float32),
        pltpu.VMEM((ZPS_DEG,), jnp.float32),
        pltpu.VMEM_SHARED((NR_DEG,), jnp.float32),
        pltpu.SemaphoreType.DMA,
    ],
)
def _deg_kernel(dst_hbm, out_hbm, idx_v, ones_v, zero_v, acc_s, sem):
    c = lax.axis_index("c")
    s = lax.axis_index("s")
    w = c * NS + s

    def fill_ones(i, carry):
        ones_v[pl.ds(i * 16, 16)] = jnp.ones((16,), jnp.float32)
        return carry

    lax.fori_loop(0, CHUNK // 16, fill_ones, 0)

    def fill_zero(i, carry):
        zero_v[pl.ds(i * 16, 16)] = jnp.zeros((16,), jnp.float32)
        return carry

    lax.fori_loop(0, ZPS_DEG // 16, fill_zero, 0)

    pltpu.sync_copy(zero_v, acc_s.at[pl.ds(s * ZPS_DEG, ZPS_DEG)])
    plsc.subcore_barrier()

    pltpu.async_copy(dst_hbm.at[w], idx_v, sem).wait()

    def body(j, carry):
        pltpu.sync_copy(ones_v, acc_s.at[idx_v.at[j]], add=True)
        return carry

    lax.fori_loop(0, CPT, body, 0)
    plsc.subcore_barrier()
    pltpu.sync_copy(acc_s.at[pl.ds(s * ZPS_DEG, ZPS_DEG)],
                    out_hbm.at[c, pl.ds(s * ZPS_DEG, ZPS_DEG)])


@functools.partial(
    pl.kernel,
    out_type=jax.ShapeDtypeStruct((NC, NROWS, H), jnp.float32),
    mesh=_sc_mesh,
    scratch_types=[
        pltpu.VMEM((4, CHUNK), jnp.int32),
        pltpu.VMEM((4, CHUNK), jnp.int32),
        pltpu.VMEM((2, CHUNK, H), jnp.float32),
        pltpu.VMEM_SHARED((NROWS, H), jnp.float32),
        pltpu.SemaphoreType.DMA,
        pltpu.SemaphoreType.DMA,
        pltpu.SemaphoreType.DMA,
        pltpu.SemaphoreType.DMA,
    ],
)
def _scatter_kernel(y_hbm, src_hbm, dst_hbm, out_hbm, sidx_v, didx_v, rows_v,
                    acc_s, si0, si1, sg0, sg1):
    c = lax.axis_index("c")
    s = lax.axis_index("s")
    w = c * NS + s

    def zero_rows(i, carry):
        for j in range(H // 16):
            rows_v[0, i, pl.ds(j * 16, 16)] = jnp.zeros((16,), jnp.float32)
        return carry

    lax.fori_loop(0, CHUNK, zero_rows, 0)
    for k in range(ZPS // CHUNK):
        pltpu.sync_copy(rows_v.at[0], acc_s.at[pl.ds(s * ZPS + k * CHUNK, CHUNK)])
    _rem = ZPS % CHUNK
    if _rem:
        pltpu.sync_copy(rows_v.at[0].at[pl.ds(0, _rem)],
                        acc_s.at[pl.ds(s * ZPS + ZPS - _rem, _rem)])
    plsc.subcore_barrier()

    def load_chunk(chunk, sem):
        pltpu.async_copy(src_hbm.at[w, chunk], sidx_v.at[chunk % 4], sem)
        pltpu.async_copy(dst_hbm.at[w, chunk], didx_v.at[chunk % 4], sem)

    def wait_chunk(chunk, sem):
        pltpu.make_async_copy(
            src_hbm.at[w, chunk], sidx_v.at[chunk % 4], sem).wait()
        pltpu.make_async_copy(
            dst_hbm.at[w, chunk], didx_v.at[chunk % 4], sem).wait()

    def start_gather(chunk, sem):
        pltpu.async_copy(
            y_hbm.at[sidx_v.at[chunk % 4]], rows_v.at[chunk % 2], sem)

    def wait_gather(chunk, sem):
        pltpu.make_async_copy(
            y_hbm.at[sidx_v.at[chunk % 4]], rows_v.at[chunk % 2], sem).wait()

    def scatter(chunk):
        pltpu.sync_copy(
            rows_v.at[chunk % 2], acc_s.at[didx_v.at[chunk % 4]], add=True)

    def pair_body(i, si_load, si_wait):
        a = 2 * i
        start_gather(a + 1, sg1)
        wait_chunk(a + 2, si_wait)
        wait_chunk(a + 3, si_wait)
        wait_gather(a, sg0)
        scatter(a)
        start_gather(a + 2, sg0)
        load_chunk(a + 4, si_load)
        wait_gather(a + 1, sg1)
        scatter(a + 1)
        load_chunk(a + 5, si_load)
        return a

    load_chunk(0, si0)
    load_chunk(1, si0)
    load_chunk(2, si1)
    load_chunk(3, si1)
    wait_chunk(0, si0)
    wait_chunk(1, si0)
    start_gather(0, sg0)

    def body(k, carry):
        i = 2 * k
        pair_body(i, si0, si1)
        pair_body(i + 1, si1, si0)
        return carry

    lax.fori_loop(0, NP // 2, body, 0)
    pair_body(NP - 1, si0, si1)
    wait_gather(CPT2, sg0)
    wait_chunk(2 * NP + 2, si0)
    wait_chunk(2 * NP + 3, si0)
    plsc.subcore_barrier()
    pltpu.sync_copy(acc_s.at[pl.ds(s * ZPS, ZPS)], out_hbm.at[c, pl.ds(s * ZPS, ZPS)])


@functools.partial(
    pl.kernel,
    out_type=jax.ShapeDtypeStruct((G, H), jnp.float32),
    mesh=_sc_mesh,
    scratch_types=[
        pltpu.VMEM((GPT,), jnp.int32),
        pltpu.VMEM((GPT, H), jnp.float32),
        pltpu.SemaphoreType.DMA,
    ],
)
def _gather_kernel(h_hbm, gi_hbm, out_hbm, idx_v, rows_v, sem):
    c = lax.axis_index("c")
    s = lax.axis_index("s")
    w = c * NS + s
    pltpu.sync_copy(gi_hbm.at[pl.ds(w * GPT, GPT)], idx_v)
    pltpu.async_copy(h_hbm.at[idx_v], rows_v, sem).wait()
    pltpu.sync_copy(rows_v, out_hbm.at[pl.ds(w * GPT, GPT)])


def _in_body(x_ref, win_ref, bin_ref, bng_ref, bnb_ref, w1_ref, d0_ref,
             d1_ref, h_ref, y_ref):
    v = jnp.dot(x_ref[...], win_ref[...], preferred_element_type=jnp.float32)
    v = v + bin_ref[...]
    h = jnp.maximum(v * (bng_ref[...] * BNS) + bnb_ref[...], 0.0)
    h_ref[...] = h
    dinv = lax.rsqrt(d0_ref[...] + d1_ref[...] + 1.0)
    y_ref[...] = dinv * jnp.dot(h, w1_ref[...], preferred_element_type=jnp.float32)


def _layer_core(h, y, p0, p1, dinv, b, bng, bnb, lng, lnb):
    agg = (p0 + p1 + y) * dinv + b
    hi = jnp.maximum(agg * (bng * BNS) + bnb, 0.0)
    t = h + hi
    mu = jnp.mean(t, axis=-1, keepdims=True)
    ctr = t - mu
    var = jnp.mean(ctr * ctr, axis=-1, keepdims=True)
    return lng * ctr * lax.rsqrt(var + EPS) + lnb


def _mid_body(h_ref, y_ref, p0_ref, p1_ref, d0_ref, d1_ref, b_ref, bng_ref,
              bnb_ref, lng_ref, lnb_ref, wn_ref, hn_ref, yn_ref):
    dinv = lax.rsqrt(d0_ref[...] + d1_ref[...] + 1.0)
    hn = _layer_core(h_ref[...], y_ref[...], p0_ref[...], p1_ref[...], dinv,
                     b_ref[...], bng_ref[...], bnb_ref[...], lng_ref[...],
                     lnb_ref[...])
    hn_ref[...] = hn
    yn_ref[...] = dinv * jnp.dot(hn, wn_ref[...], preferred_element_type=jnp.float32)


def _last_body(h_ref, y_ref, p0_ref, p1_ref, d0_ref, d1_ref, b_ref, bng_ref,
               bnb_ref, lng_ref, lnb_ref, hn_ref):
    dinv = lax.rsqrt(d0_ref[...] + d1_ref[...] + 1.0)
    hn_ref[...] = _layer_core(h_ref[...], y_ref[...], p0_ref[...], p1_ref[...],
                              dinv, b_ref[...], bng_ref[...], bnb_ref[...],
                              lng_ref[...], lnb_ref[...])


def _head_body(xg_ref, w1_ref, b1_ref, g_ref, bb_ref, w2_ref, b2_ref, w3_ref,
               b3_ref, o_ref):
    z = jnp.dot(xg_ref[...], w1_ref[...], preferred_element_type=jnp.float32)
    z = jnp.maximum((z + b1_ref[...]) * (g_ref[...] * BNS) + bb_ref[...], 0.0)
    z = jnp.maximum(
        jnp.dot(z, w2_ref[...], preferred_element_type=jnp.float32) + b2_ref[...], 0.0)
    z = jnp.dot(z, w3_ref[...], preferred_element_type=jnp.float32) + b3_ref[...]
    m = jnp.max(z, axis=-1, keepdims=True)
    ez = jnp.exp(z - m)
    o_ref[...] = z - m - jnp.log(jnp.sum(ez, axis=-1, keepdims=True))


def _row_spec(shape):
    return pl.BlockSpec(shape, lambda i: (i, 0))


def _full_spec(shape):
    return pl.BlockSpec(shape, lambda i: (0, 0))


_in_call = pl.pallas_call(
    _in_body,
    grid=(NBLK,),
    in_specs=[
        _row_spec((RB, D)), _full_spec((D, H)), _full_spec((1, H)),
        _full_spec((1, H)), _full_spec((1, H)), _full_spec((H, H)),
        _row_spec((RB, 1)), _row_spec((RB, 1)),
    ],
    out_specs=[_row_spec((RB, H)), _row_spec((RB, H))],
    out_shape=[jax.ShapeDtypeStruct((N, H), jnp.float32),
               jax.ShapeDtypeStruct((N, H), jnp.float32)],
)

_mid_call = pl.pallas_call(
    _mid_body,
    grid=(NBLK,),
    in_specs=[
        _row_spec((RB, H)), _row_spec((RB, H)), _row_spec((RB, H)),
        _row_spec((RB, H)), _row_spec((RB, 1)), _row_spec((RB, 1)),
        _full_spec((1, H)), _full_spec((1, H)), _full_spec((1, H)),
        _full_spec((1, H)), _full_spec((1, H)), _full_spec((H, H)),
    ],
    out_specs=[_row_spec((RB, H)), _row_spec((RB, H))],
    out_shape=[jax.ShapeDtypeStruct((N, H), jnp.float32),
               jax.ShapeDtypeStruct((N, H), jnp.float32)],
)

_last_call = pl.pallas_call(
    _last_body,
    grid=(NBLK,),
    in_specs=[
        _row_spec((RB, H)), _row_spec((RB, H)), _row_spec((RB, H)),
        _row_spec((RB, H)), _row_spec((RB, 1)), _row_spec((RB, 1)),
        _full_spec((1, H)), _full_spec((1, H)), _full_spec((1, H)),
        _full_spec((1, H)), _full_spec((1, H)),
    ],
    out_specs=[_row_spec((RB, H))],
    out_shape=[jax.ShapeDtypeStruct((N, H), jnp.float32)],
)

_head_call = pl.pallas_call(
    _head_body,
    grid=(1,),
    in_specs=[
        _full_spec((G, H)), _full_spec((H, H // 2)), _full_spec((1, H // 2)),
        _full_spec((1, H // 2)), _full_spec((1, H // 2)),
        _full_spec((H // 2, H // 4)), _full_spec((1, H // 4)),
        _full_spec((H // 4, 2)), _full_spec((1, 2)),
    ],
    out_specs=[_full_spec((G, 2))],
    out_shape=[jax.ShapeDtypeStruct((G, 2), jnp.float32)],
)


def kernel(x, edge_index, game_indices, params):
    p = params
    src = edge_index[0]
    dst = edge_index[1]
    pad = EPAD - E
    trash = N + (jnp.arange(pad, dtype=jnp.int32) % (NROWS - N))
    dum_trash = N + (
        jnp.arange(NW * NDUM * CHUNK, dtype=jnp.int32) % (NROWS - N)
    ).reshape(NW, NDUM, CHUNK)
    srcp = jnp.concatenate([src, jnp.zeros((pad,), jnp.int32)]).reshape(NW, CPT, CHUNK)
    dstp = jnp.concatenate([dst, trash]).reshape(NW, CPT, CHUNK)
    srcp = jnp.concatenate(
        [srcp, jnp.zeros((NW, NDUM, CHUNK), jnp.int32)], axis=1)
    dstp = jnp.concatenate([dstp, dum_trash], axis=1)

    pdeg = _deg_kernel(dstp)
    d0 = pdeg[0, :N].reshape(N, 1)
    d1 = pdeg[1, :N].reshape(N, 1)

    def rv(name):
        return p[name].reshape(1, -1)

    h, y = _in_call(x, p['W_in'], rv('b_in'), rv('bn_in_g'), rv('bn_in_b'),
                    p['W1'], d0, d1)

    for i in (1, 2, 3):
        part = _scatter_kernel(y, srcp, dstp)
        p0 = part[0, :N]
        p1 = part[1, :N]
        bn_args = (rv('b%d' % i), rv('bn%d_g' % i), rv('bn%d_b' % i),
                   rv('ln%d_g' % i), rv('ln%d_b' % i))
        if i < 3:
            h, y = _mid_call(h, y, p0, p1, d0, d1, *bn_args, p['W%d' % (i + 1)])
        else:
            (h,) = _last_call(h, y, p0, p1, d0, d1, *bn_args)

    xg = _gather_kernel(h, game_indices)
    (out,) = _head_call(xg, p['fc1_W'], rv('fc1_b'), rv('fc_bn_g'),
                        rv('fc_bn_b'), p['fc2_W'], rv('fc2_b'), p['fc3_W'],
                        rv('fc3_b'))
    return out


# speedup vs baseline: 3.7425x; 3.7425x over previous
import functools

import jax
import jax.numpy as jnp
from jax import lax
from jax.experimental import pallas as pl
from jax.experimental.pallas import tpu as pltpu
from jax.experimental.pallas import tpu_sc as plsc

N = 10000
D = 128
H = 128
E = 320000
G = 1024
EPS = 1e-5
BNS = (1.0 + EPS) ** -0.5

NC = 2
NS = 16
NW = NC * NS
CHUNK = 128
CPT0 = 102
CPT1 = 55
CPT_MAX = max(CPT0, CPT1)
CAP0 = NS * CPT0 * CHUNK
CAP1 = NS * CPT1 * CHUNK
NROWS = 10112
ZPS = NROWS // NS
NR_DEG = 10240
ZPS_DEG = NR_DEG // NS
GPT = G // NW

RB = 2000
NBLK = N // RB

_sc_mesh = plsc.VectorSubcoreMesh(
    core_axis_name="c", subcore_axis_name="s", num_cores=NC, num_subcores=NS
)


@functools.partial(
    pl.kernel,
    out_type=jax.ShapeDtypeStruct((NC, NR_DEG), jnp.float32),
    mesh=_sc_mesh,
    scratch_types=[
        pltpu.VMEM((CPT_MAX, CHUNK), jnp.int32),
        pltpu.VMEM((CHUNK,), jnp.float32),
        pltpu.VMEM((ZPS_DEG,), jnp.float32),
        pltpu.VMEM_SHARED((NR_DEG,), jnp.float32),
        pltpu.SemaphoreType.DMA,
    ],
)
def _deg_kernel(dst_hbm, out_hbm, idx_v, ones_v, zero_v, acc_s, sem):
    c = lax.axis_index("c")
    s = lax.axis_index("s")
    w = c * NS + s
    cnt = jnp.where(c == 0, CPT0, CPT1)

    def fill_ones(i, carry):
        ones_v[pl.ds(i * 16, 16)] = jnp.ones((16,), jnp.float32)
        return carry

    lax.fori_loop(0, CHUNK // 16, fill_ones, 0)

    def fill_zero(i, carry):
        zero_v[pl.ds(i * 16, 16)] = jnp.zeros((16,), jnp.float32)
        return carry

    lax.fori_loop(0, ZPS_DEG // 16, fill_zero, 0)

    pltpu.sync_copy(zero_v, acc_s.at[pl.ds(s * ZPS_DEG, ZPS_DEG)])
    plsc.subcore_barrier()

    pltpu.async_copy(dst_hbm.at[w], idx_v, sem).wait()

    def body(j, carry):
        pltpu.sync_copy(ones_v, acc_s.at[idx_v.at[j]], add=True)
        return carry

    lax.fori_loop(0, cnt, body, 0)
    plsc.subcore_barrier()
    pltpu.sync_copy(acc_s.at[pl.ds(s * ZPS_DEG, ZPS_DEG)],
                    out_hbm.at[c, pl.ds(s * ZPS_DEG, ZPS_DEG)])


@functools.partial(
    pl.kernel,
    out_type=jax.ShapeDtypeStruct((NC, NROWS, H), jnp.float32),
    mesh=_sc_mesh,
    scratch_types=[
        pltpu.VMEM((CPT_MAX, CHUNK), jnp.int32),
        pltpu.VMEM((CPT_MAX, CHUNK), jnp.int32),
        pltpu.VMEM((CHUNK, H), jnp.float32),
        pltpu.VMEM_SHARED((NROWS, H), jnp.float32),
        pltpu.SemaphoreType.DMA,
        pltpu.SemaphoreType.DMA,
    ],
)
def _scatter_kernel(y_hbm, src_hbm, dst_hbm, out_hbm, src_v, dst_v, rows_v,
                    acc_s, sem_i, sem_g):
    c = lax.axis_index("c")
    s = lax.axis_index("s")
    w = c * NS + s
    cnt = jnp.where(c == 0, CPT0, CPT1)

    def zero_rows(i, carry):
        for j in range(H // 16):
            rows_v[i, pl.ds(j * 16, 16)] = jnp.zeros((16,), jnp.float32)
        return carry

    lax.fori_loop(0, CHUNK, zero_rows, 0)
    for k in range(ZPS // CHUNK):
        pltpu.sync_copy(rows_v, acc_s.at[pl.ds(s * ZPS + k * CHUNK, CHUNK)])
    _rem = ZPS % CHUNK
    if _rem:
        pltpu.sync_copy(rows_v.at[pl.ds(0, _rem)],
                        acc_s.at[pl.ds(s * ZPS + ZPS - _rem, _rem)])
    plsc.subcore_barrier()

    pltpu.async_copy(src_hbm.at[w], src_v, sem_i).wait()
    pltpu.async_copy(dst_hbm.at[w], dst_v, sem_i).wait()

    def body(j, carry):
        pltpu.async_copy(y_hbm.at[src_v.at[j]], rows_v, sem_g).wait()
        pltpu.sync_copy(rows_v, acc_s.at[dst_v.at[j]], add=True)
        return carry

    lax.fori_loop(0, cnt, body, 0)
    plsc.subcore_barrier()
    pltpu.sync_copy(acc_s.at[pl.ds(s * ZPS, ZPS)], out_hbm.at[c, pl.ds(s * ZPS, ZPS)])


@functools.partial(
    pl.kernel,
    out_type=jax.ShapeDtypeStruct((G, H), jnp.float32),
    mesh=_sc_mesh,
    scratch_types=[
        pltpu.VMEM((GPT,), jnp.int32),
        pltpu.VMEM((GPT, H), jnp.float32),
        pltpu.SemaphoreType.DMA,
    ],
)
def _gather_kernel(h_hbm, gi_hbm, out_hbm, idx_v, rows_v, sem):
    c = lax.axis_index("c")
    s = lax.axis_index("s")
    w = c * NS + s
    pltpu.sync_copy(gi_hbm.at[pl.ds(w * GPT, GPT)], idx_v)
    pltpu.async_copy(h_hbm.at[idx_v], rows_v, sem).wait()
    pltpu.sync_copy(rows_v, out_hbm.at[pl.ds(w * GPT, GPT)])


def _in_body(x_ref, win_ref, bin_ref, bng_ref, bnb_ref, w1_ref, d0_ref,
             d1_ref, h_ref, y_ref):
    v = jnp.dot(x_ref[...], win_ref[...], preferred_element_type=jnp.float32)
    v = v + bin_ref[...]
    h = jnp.maximum(v * (bng_ref[...] * BNS) + bnb_ref[...], 0.0)
    h_ref[...] = h
    dinv = lax.rsqrt(d0_ref[...] + d1_ref[...] + 1.0)
    y_ref[...] = dinv * jnp.dot(h, w1_ref[...], preferred_element_type=jnp.float32)


def _layer_core(h, y, p0, p1, dinv, b, bng, bnb, lng, lnb):
    agg = (p0 + p1 + y) * dinv + b
    hi = jnp.maximum(agg * (bng * BNS) + bnb, 0.0)
    t = h + hi
    mu = jnp.mean(t, axis=-1, keepdims=True)
    ctr = t - mu
    var = jnp.mean(ctr * ctr, axis=-1, keepdims=True)
    return lng * ctr * lax.rsqrt(var + EPS) + lnb


def _mid_body(h_ref, y_ref, p0_ref, p1_ref, d0_ref, d1_ref, b_ref, bng_ref,
              bnb_ref, lng_ref, lnb_ref, wn_ref, hn_ref, yn_ref):
    dinv = lax.rsqrt(d0_ref[...] + d1_ref[...] + 1.0)
    hn = _layer_core(h_ref[...], y_ref[...], p0_ref[...], p1_ref[...], dinv,
                     b_ref[...], bng_ref[...], bnb_ref[...], lng_ref[...],
                     lnb_ref[...])
    hn_ref[...] = hn
    yn_ref[...] = dinv * jnp.dot(hn, wn_ref[...], preferred_element_type=jnp.float32)


def _last_body(h_ref, y_ref, p0_ref, p1_ref, d0_ref, d1_ref, b_ref, bng_ref,
               bnb_ref, lng_ref, lnb_ref, hn_ref):
    dinv = lax.rsqrt(d0_ref[...] + d1_ref[...] + 1.0)
    hn_ref[...] = _layer_core(h_ref[...], y_ref[...], p0_ref[...], p1_ref[...],
                              dinv, b_ref[...], bng_ref[...], bnb_ref[...],
                              lng_ref[...], lnb_ref[...])


def _head_body(xg_ref, w1_ref, b1_ref, g_ref, bb_ref, w2_ref, b2_ref, w3_ref,
               b3_ref, o_ref):
    z = jnp.dot(xg_ref[...], w1_ref[...], preferred_element_type=jnp.float32)
    z = jnp.maximum((z + b1_ref[...]) * (g_ref[...] * BNS) + bb_ref[...], 0.0)
    z = jnp.maximum(
        jnp.dot(z, w2_ref[...], preferred_element_type=jnp.float32) + b2_ref[...], 0.0)
    z = jnp.dot(z, w3_ref[...], preferred_element_type=jnp.float32) + b3_ref[...]
    m = jnp.max(z, axis=-1, keepdims=True)
    ez = jnp.exp(z - m)
    o_ref[...] = z - m - jnp.log(jnp.sum(ez, axis=-1, keepdims=True))


def _row_spec(shape):
    return pl.BlockSpec(shape, lambda i: (i, 0))


def _full_spec(shape):
    return pl.BlockSpec(shape, lambda i: (0, 0))


_in_call = pl.pallas_call(
    _in_body,
    grid=(NBLK,),
    in_specs=[
        _row_spec((RB, D)), _full_spec((D, H)), _full_spec((1, H)),
        _full_spec((1, H)), _full_spec((1, H)), _full_spec((H, H)),
        _row_spec((RB, 1)), _row_spec((RB, 1)),
    ],
    out_specs=[_row_spec((RB, H)), _row_spec((RB, H))],
    out_shape=[jax.ShapeDtypeStruct((N, H), jnp.float32),
               jax.ShapeDtypeStruct((N, H), jnp.float32)],
)

_mid_call = pl.pallas_call(
    _mid_body,
    grid=(NBLK,),
    in_specs=[
        _row_spec((RB, H)), _row_spec((RB, H)), _row_spec((RB, H)),
        _row_spec((RB, H)), _row_spec((RB, 1)), _row_spec((RB, 1)),
        _full_spec((1, H)), _full_spec((1, H)), _full_spec((1, H)),
        _full_spec((1, H)), _full_spec((1, H)), _full_spec((H, H)),
    ],
    out_specs=[_row_spec((RB, H)), _row_spec((RB, H))],
    out_shape=[jax.ShapeDtypeStruct((N, H), jnp.float32),
               jax.ShapeDtypeStruct((N, H), jnp.float32)],
)

_last_call = pl.pallas_call(
    _last_body,
    grid=(NBLK,),
    in_specs=[
        _row_spec((RB, H)), _row_spec((RB, H)), _row_spec((RB, H)),
        _row_spec((RB, H)), _row_spec((RB, 1)), _row_spec((RB, 1)),
        _full_spec((1, H)), _full_spec((1, H)), _full_spec((1, H)),
        _full_spec((1, H)), _full_spec((1, H)),
    ],
    out_specs=[_row_spec((RB, H))],
    out_shape=[jax.ShapeDtypeStruct((N, H), jnp.float32)],
)

_head_call = pl.pallas_call(
    _head_body,
    grid=(1,),
    in_specs=[
        _full_spec((G, H)), _full_spec((H, H // 2)), _full_spec((1, H // 2)),
        _full_spec((1, H // 2)), _full_spec((1, H // 2)),
        _full_spec((H // 2, H // 4)), _full_spec((1, H // 4)),
        _full_spec((H // 4, 2)), _full_spec((1, 2)),
    ],
    out_specs=[_full_spec((G, 2))],
    out_shape=[jax.ShapeDtypeStruct((G, 2), jnp.float32)],
)


def kernel(x, edge_index, game_indices, params):
    p = params
    src = edge_index[0]
    dst = edge_index[1]
    pad = CAP0 + CAP1 - E
    trash = N + (jnp.arange(pad, dtype=jnp.int32) % (NROWS - N))
    srcf = jnp.concatenate([src, jnp.zeros((pad,), jnp.int32)])
    dstf = jnp.concatenate([dst, trash])

    def split(flat, fillval):
        c0 = flat[:CAP0].reshape(NS, CPT0, CHUNK)
        c1 = flat[CAP0:].reshape(NS, CPT1, CHUNK)
        c1 = jnp.concatenate(
            [c1, jnp.full((NS, CPT_MAX - CPT1, CHUNK), fillval, jnp.int32)],
            axis=1)
        if CPT_MAX > CPT0:
            c0 = jnp.concatenate(
                [c0, jnp.full((NS, CPT_MAX - CPT0, CHUNK), fillval, jnp.int32)],
                axis=1)
        return jnp.concatenate([c0, c1], axis=0)

    srcp = split(srcf, 0)
    dstp = split(dstf, N)

    pdeg = _deg_kernel(dstp)
    d0 = pdeg[0, :N].reshape(N, 1)
    d1 = pdeg[1, :N].reshape(N, 1)

    def rv(name):
        return p[name].reshape(1, -1)

    h, y = _in_call(x, p['W_in'], rv('b_in'), rv('bn_in_g'), rv('bn_in_b'),
                    p['W1'], d0, d1)

    for i in (1, 2, 3):
        part = _scatter_kernel(y, srcp, dstp)
        p0 = part[0, :N]
        p1 = part[1, :N]
        bn_args = (rv('b%d' % i), rv('bn%d_g' % i), rv('bn%d_b' % i),
                   rv('ln%d_g' % i), rv('ln%d_b' % i))
        if i < 3:
            h, y = _mid_call(h, y, p0, p1, d0, d1, *bn_args, p['W%d' % (i + 1)])
        else:
            (h,) = _last_call(h, y, p0, p1, d0, d1, *bn_args)

    xg = _gather_kernel(h, game_indices)
    (out,) = _head_call(xg, p['fc1_W'], rv('fc1_b'), rv('fc_bn_g'),
                        rv('fc_bn_b'), p['fc2_W'], rv('fc2_b'), p['fc3_W'],
                        rv('fc3_b'))
    return out
